# Initial kernel scaffold; baseline (speedup 1.0000x reference)
import jax
import jax.numpy as jnp
from jax.experimental import pallas as pl


def kernel(activations):
    raise NotImplementedError("write your pallas kernel here")


# speedup vs baseline: 11.7776x; 11.7776x over previous
import math

import jax
import jax.numpy as jnp
import numpy as np
from jax.experimental import pallas as pl
from jax.experimental.pallas import tpu as pltpu

_LMAX = 5
_TAU = 8
_BATCH_BLOCK = 128


def _fct(n):
    return math.factorial(n)


def _cg_coef(l1, l2, l, m1, m2, m):
    if m1 + m2 != m:
        return 0.0
    if l < abs(l1 - l2) or l > l1 + l2:
        return 0.0
    pre = math.sqrt((2 * l + 1) * _fct(l + l1 - l2) * _fct(l - l1 + l2)
                    * _fct(l1 + l2 - l) / _fct(l1 + l2 + l + 1))
    pre *= math.sqrt(_fct(l + m) * _fct(l - m) * _fct(l1 - m1) * _fct(l1 + m1)
                     * _fct(l2 - m2) * _fct(l2 + m2))
    kmin = max(0, l2 - l - m1, l1 - l + m2)
    kmax = min(l1 + l2 - l, l1 - m1, l2 + m2)
    s = 0.0
    for k in range(kmin, kmax + 1):
        s += (-1.0) ** k / (_fct(k) * _fct(l1 + l2 - l - k) * _fct(l1 - m1 - k)
                            * _fct(l2 + m2 - k) * _fct(l - l2 + m1 + k)
                            * _fct(l - l1 - m2 + k))
    return pre * s


def _build_plan():
    degs = [_TAU * (2 * l + 1) for l in range(_LMAX + 1)]
    cum = [0]
    for d in degs:
        cum.append(cum[-1] + d)
    base = {}
    off = 0
    for l in range(_LMAX + 1):
        for l1 in range(_LMAX + 1):
            for l2 in range(l1 + 1):
                if abs(l1 - l2) <= l <= l1 + l2:
                    base[(l, l1, l2)] = off
                    off += _TAU * _TAU * (2 * l + 1)
    pairs = []
    for l1 in range(_LMAX + 1):
        for l2 in range(l1 + 1):
            ls = [l for l in range(_LMAX + 1) if abs(l1 - l2) <= l <= l1 + l2]
            if not ls:
                continue
            pq_terms = []
            for p in range(2 * l1 + 1):
                for q in range(2 * l2 + 1):
                    m1, m2 = p - l1, q - l2
                    m = m1 + m2
                    outs = []
                    for l in ls:
                        if abs(m) <= l:
                            c = _cg_coef(l1, l2, l, m1, m2, m)
                            if abs(c) > 1e-12:
                                outs.append((l, m + l, float(c)))
                    if outs:
                        pq_terms.append((p, q, outs))
            pairs.append(dict(l1=l1, l2=l2, c1=cum[l1], c2=cum[l2],
                              ls=ls, pq=pq_terms,
                              bases={l: base[(l, l1, l2)] for l in ls}))
    return pairs, cum[-1], off


_PAIRS, _IN_LEN, _OUT_LEN = _build_plan()


def _cg_kernel(fr_ref, fi_ref, out_ref):
    nb = _BATCH_BLOCK
    fr = fr_ref[...]
    fi = fi_ref[...]
    for pair in _PAIRS:
        p1 = 2 * pair["l1"] + 1
        p2 = 2 * pair["l2"] + 1
        f1r = fr[pair["c1"]:pair["c1"] + _TAU * p1, :].reshape(_TAU, p1, nb)
        f1i = fi[pair["c1"]:pair["c1"] + _TAU * p1, :].reshape(_TAU, p1, nb)
        f2r = fr[pair["c2"]:pair["c2"] + _TAU * p2, :].reshape(_TAU, p2, nb)
        f2i = fi[pair["c2"]:pair["c2"] + _TAU * p2, :].reshape(_TAU, p2, nb)
        acc_r = {l: [[] for _ in range(2 * l + 1)] for l in pair["ls"]}
        acc_i = {l: [[] for _ in range(2 * l + 1)] for l in pair["ls"]}
        for (p, q, outs) in pair["pq"]:
            a_r = f1r[:, p, :][:, None, :]
            a_i = f1i[:, p, :][:, None, :]
            b_r = f2r[:, q, :][None, :, :]
            b_i = f2i[:, q, :][None, :, :]
            t_r = a_r * b_r - a_i * b_i
            t_i = a_r * b_i + a_i * b_r
            for (l, x, c) in outs:
                acc_r[l][x].append(c * t_r)
                acc_i[l][x].append(c * t_i)
        for l in pair["ls"]:
            X = 2 * l + 1
            zeros = jnp.zeros((_TAU, _TAU, nb), dtype=jnp.float32)
            rows = []
            for x in range(X):
                sr = sum(acc_r[l][x]) if acc_r[l][x] else zeros
                si = sum(acc_i[l][x]) if acc_i[l][x] else zeros
                rows.append((sr, si))
            r_stack = jnp.stack([sr for sr, _ in rows], axis=2)
            i_stack = jnp.stack([si for _, si in rows], axis=2)
            inter = jnp.stack([r_stack, i_stack], axis=3)
            inter = inter.reshape(_TAU * _TAU * X * 2, nb)
            b0 = 2 * pair["bases"][l]
            out_ref[:, b0:b0 + 2 * _TAU * _TAU * X] = inter.T


def kernel(activations):
    B = activations.shape[0]
    fr = activations[..., 0].T
    fi = activations[..., 1].T
    grid = (B // _BATCH_BLOCK,)
    out2 = pl.pallas_call(
        _cg_kernel,
        grid=grid,
        in_specs=[
            pl.BlockSpec((_IN_LEN, _BATCH_BLOCK), lambda i: (0, i)),
            pl.BlockSpec((_IN_LEN, _BATCH_BLOCK), lambda i: (0, i)),
        ],
        out_specs=pl.BlockSpec((_BATCH_BLOCK, 2 * _OUT_LEN), lambda i: (i, 0)),
        out_shape=jax.ShapeDtypeStruct((B, 2 * _OUT_LEN), jnp.float32),
        compiler_params=pltpu.CompilerParams(
            vmem_limit_bytes=100 * 1024 * 1024),
    )(fr, fi)
    return out2.reshape(B, _OUT_LEN, 2)
